# baseline (speedup 1.0000x reference)
import functools

import jax
import jax.numpy as jnp
from jax import lax
from jax.experimental import pallas as pl
from jax.experimental.pallas import tpu as pltpu
from jax.experimental.pallas import tpu_sc as plsc

EMB = 64
T = 200
B = 1024

NUM_CORES = 2
NUM_SUBCORES = 16
NUM_WORKERS = NUM_CORES * NUM_SUBCORES
ROWS_PER_WORKER = B // NUM_WORKERS
IDX_SPLIT = 2
IDX_MINOR = T // IDX_SPLIT
LANES = 16


def _emb_body(x_hbm, tok_hbm, pos_hbm, out_hbm, idx_v, rows_v, pos_v, sem):
    wid = lax.axis_index("s") * NUM_CORES + lax.axis_index("c")
    pltpu.sync_copy(pos_hbm, pos_v)

    def batch_body(c, carry):
        b = wid * ROWS_PER_WORKER + c
        pltpu.sync_copy(x_hbm.at[b], idx_v)
        cp0 = pltpu.async_copy(
            tok_hbm.at[idx_v.at[0]], rows_v.at[pl.ds(0, IDX_MINOR)], sem)
        cp1 = pltpu.async_copy(
            tok_hbm.at[idx_v.at[1]], rows_v.at[pl.ds(IDX_MINOR, IDX_MINOR)], sem)
        cp0.wait()
        cp1.wait()

        def row_body(r, carry2):
            for j in range(EMB // LANES):
                sl = pl.ds(j * LANES, LANES)
                rows_v[r, sl] = rows_v[r, sl] + pos_v[r, sl]
            return carry2

        lax.fori_loop(0, T, row_body, 0)
        pltpu.sync_copy(rows_v, out_hbm.at[b])
        return carry

    lax.fori_loop(0, ROWS_PER_WORKER, batch_body, 0)


@jax.jit
def kernel(x, token_table, pos_table):
    x3 = x.astype(jnp.int32).reshape(B, IDX_SPLIT, IDX_MINOR)
    mesh = plsc.VectorSubcoreMesh(core_axis_name="c", subcore_axis_name="s")
    run = functools.partial(
        pl.kernel,
        mesh=mesh,
        out_type=jax.ShapeDtypeStruct((B, T, EMB), jnp.float32),
        scratch_types=[
            pltpu.VMEM((IDX_SPLIT, IDX_MINOR), jnp.int32),
            pltpu.VMEM((T, EMB), jnp.float32),
            pltpu.VMEM((T, EMB), jnp.float32),
            pltpu.SemaphoreType.DMA,
        ],
        compiler_params=pltpu.CompilerParams(use_tc_tiling_on_sc=False),
    )(_emb_body)
    return run(x3, token_table, pos_table)
